# Initial kernel scaffold; baseline (speedup 1.0000x reference)
import jax
import jax.numpy as jnp
from jax.experimental import pallas as pl


def kernel(x, sim_matrix, gates_param, fc1_w, fc1_b, fc2_w, fc2_b):
    raise NotImplementedError("write your pallas kernel here")


# speedup vs baseline: 1.1581x; 1.1581x over previous
import jax
import jax.numpy as jnp
from jax.experimental import pallas as pl
from jax.experimental.pallas import tpu as pltpu

E = 8
D = 1024
H = 4096
T = 2048

BH = 512
NH = H // BH


def _ffn_body(x_ref, m_ref, s_ref, w1_ref, b1_ref, w2_ref, b2_ref,
              out_ref, laux_ref, acc_ref):
    e = pl.program_id(0)
    h = pl.program_id(1)

    @pl.when(h == 0)
    def _():
        acc_ref[...] = jnp.zeros_like(acc_ref)

    t1 = jnp.dot(x_ref[...], w1_ref[0], preferred_element_type=jnp.float32)
    t1 = jnp.maximum(t1 + b1_ref[0], 0.0)
    acc_ref[...] += jnp.dot(t1, w2_ref[0], preferred_element_type=jnp.float32)

    @pl.when(h == NH - 1)
    def _():
        m = m_ref[0]
        contrib = m * (acc_ref[...] + b2_ref[0])
        part = (jnp.float32(E) * jnp.mean(s_ref[0]) * jnp.mean(m)).reshape(1, 1)

        @pl.when(e == 0)
        def _():
            out_ref[...] = contrib
            laux_ref[...] = part

        @pl.when(e > 0)
        def _():
            out_ref[...] += contrib
            laux_ref[...] += part


def _normed(v, axis):
    n = jnp.sqrt(jnp.sum(v * v, axis=axis, keepdims=True))
    return v / jnp.maximum(n, 1e-12)


def kernel(x, sim_matrix, gates_param, fc1_w, fc1_b, fc2_w, fc2_b):
    logits = jax.nn.sigmoid(
        jnp.matmul(_normed(x, 1), _normed(sim_matrix[:, :E], 0)))
    gate_thresh = jax.nn.sigmoid(gates_param[:E])
    scores = jax.nn.relu(logits - gate_thresh)
    signed = scores + jax.lax.stop_gradient(jnp.sign(scores) - scores)

    maskc = signed.T.reshape(E, T, 1)
    scoresc = scores.T.reshape(E, T, 1)

    out, laux = pl.pallas_call(
        _ffn_body,
        grid=(E, NH),
        out_shape=(
            jax.ShapeDtypeStruct((T, D), jnp.float32),
            jax.ShapeDtypeStruct((1, 1), jnp.float32),
        ),
        in_specs=[
            pl.BlockSpec((T, D), lambda e, h: (0, 0)),
            pl.BlockSpec((1, T, 1), lambda e, h: (e, 0, 0)),
            pl.BlockSpec((1, T, 1), lambda e, h: (e, 0, 0)),
            pl.BlockSpec((1, D, BH), lambda e, h: (e, 0, h)),
            pl.BlockSpec((1, 1, BH), lambda e, h: (e, 0, h)),
            pl.BlockSpec((1, BH, D), lambda e, h: (e, h, 0)),
            pl.BlockSpec((1, 1, D), lambda e, h: (e, 0, 0)),
        ],
        out_specs=(
            pl.BlockSpec((T, D), lambda e, h: (0, 0)),
            pl.BlockSpec((1, 1), lambda e, h: (0, 0)),
        ),
        scratch_shapes=[pltpu.VMEM((T, D), jnp.float32)],
    )(x, maskc, scoresc, fc1_w, fc1_b, fc2_w, fc2_b)

    return (out, laux[0, 0])
